# Initial kernel scaffold; baseline (speedup 1.0000x reference)
import jax
import jax.numpy as jnp
from jax.experimental import pallas as pl


def kernel(mask, edge_weight, xs, ys):
    raise NotImplementedError("write your pallas kernel here")


# speedup vs baseline: 91.4689x; 91.4689x over previous
import functools

import jax
import jax.numpy as jnp
from jax import lax
from jax.experimental import pallas as pl
from jax.experimental.pallas import tpu as pltpu
from jax.experimental.pallas import tpu_sc as plsc

N = 4096
TOTAL = N * (N + 1) // 2
NC = 2
NS = 16
NW = NC * NS
ROWS_PER_W = N // NW
CH = 4128


def _densify_body(ew_hbm, l_hbm, chunk, row_v, sem):
    wid = lax.axis_index("s") * NC + lax.axis_index("c")
    iota16 = lax.iota(jnp.int32, 16)

    def row_body(k, _):
        i = wid + NW * k
        t = (i * (i + 1)) // 2
        s0 = t - lax.rem(t, 8)
        s = jnp.minimum(s0, TOTAL - CH)
        s = pl.multiple_of(s, 8)
        r = t - s
        pltpu.async_copy(ew_hbm.at[pl.ds(s, CH)], chunk, sem).wait()
        n16 = (i + 16) // 16

        def shift_body(q, _):
            v = plsc.load_gather(chunk, [r + 16 * q + iota16])
            row_v[pl.ds(16 * q, 16)] = v
            return 0

        lax.fori_loop(0, n16, shift_body, 0)
        pltpu.async_copy(row_v, l_hbm.at[i], sem).wait()
        return 0

    lax.fori_loop(0, ROWS_PER_W, row_body, 0)


def _densify(edge_weight):
    mesh = plsc.VectorSubcoreMesh(
        core_axis_name="c", subcore_axis_name="s", num_cores=NC, num_subcores=NS
    )
    return pl.kernel(
        _densify_body,
        out_type=jax.ShapeDtypeStruct((N, N), jnp.float32),
        mesh=mesh,
        compiler_params=pltpu.CompilerParams(needs_layout_passes=False),
        scratch_types=[
            pltpu.VMEM((CH,), jnp.float32),
            pltpu.VMEM((N,), jnp.float32),
            pltpu.SemaphoreType.DMA,
        ],
    )(edge_weight)


B = 256


def _sym_body(l_ref, m_ref, o_ref):
    bi = pl.program_id(0)
    bj = pl.program_id(1)
    lb = l_ref[...]

    @pl.when(bi > bj)
    def _():
        o_ref[...] = lb * m_ref[...]

    @pl.when(bi < bj)
    def _():
        o_ref[...] = lb.T * m_ref[...]

    @pl.when(bi == bj)
    def _():
        rr = lax.broadcasted_iota(jnp.int32, (B, B), 0)
        cc = lax.broadcasted_iota(jnp.int32, (B, B), 1)
        o_ref[...] = jnp.where(cc <= rr, lb, lb.T) * m_ref[...]


def _symmetrize_mask(l_mat, mask):
    grid = (N // B, N // B)
    return pl.pallas_call(
        _sym_body,
        grid=grid,
        in_specs=[
            pl.BlockSpec((B, B), lambda i, j: (jnp.maximum(i, j), jnp.minimum(i, j))),
            pl.BlockSpec((B, B), lambda i, j: (i, j)),
        ],
        out_specs=pl.BlockSpec((B, B), lambda i, j: (i, j)),
        out_shape=jax.ShapeDtypeStruct((N, N), jnp.float32),
    )(l_mat, mask)


@jax.jit
def kernel(mask, edge_weight, xs, ys):
    l_mat = _densify(edge_weight)
    out = _symmetrize_mask(l_mat, mask)
    return out.reshape(-1)


# speedup vs baseline: 133.9124x; 1.4640x over previous
import functools

import jax
import jax.numpy as jnp
from jax import lax
from jax.experimental import pallas as pl
from jax.experimental.pallas import tpu as pltpu
from jax.experimental.pallas import tpu_sc as plsc

N = 4096
TOTAL = N * (N + 1) // 2
NC = 2
NS = 16
NW = NC * NS
ROWS_PER_W = N // NW
CH = 4128


def _densify_body(ew_hbm, l_hbm, chunk, row_v, sem):
    wid = lax.axis_index("s") * NC + lax.axis_index("c")
    iota16 = lax.iota(jnp.int32, 16)

    def row_body(k, _):
        i = wid + NW * k
        t = (i * (i + 1)) // 2
        s0 = t - lax.rem(t, 8)
        s = jnp.minimum(s0, TOTAL - CH)
        s = pl.multiple_of(s, 8)
        r = t - s
        pltpu.async_copy(ew_hbm.at[pl.ds(s, CH)], chunk, sem).wait()
        n16 = (i + 16) // 16

        def shift_body(q, _):
            v = plsc.load_gather(chunk, [r + 16 * q + iota16])
            row_v[pl.ds(16 * q, 16)] = v
            return 0

        lax.fori_loop(0, n16, shift_body, 0)
        pltpu.async_copy(row_v, l_hbm.at[i], sem).wait()
        return 0

    lax.fori_loop(0, ROWS_PER_W, row_body, 0)


def _densify(edge_weight):
    mesh = plsc.VectorSubcoreMesh(
        core_axis_name="c", subcore_axis_name="s", num_cores=NC, num_subcores=NS
    )
    return pl.kernel(
        _densify_body,
        out_type=jax.ShapeDtypeStruct((N, N), jnp.float32),
        mesh=mesh,
        compiler_params=pltpu.CompilerParams(needs_layout_passes=False),
        scratch_types=[
            pltpu.VMEM((CH,), jnp.float32),
            pltpu.VMEM((N,), jnp.float32),
            pltpu.SemaphoreType.DMA,
        ],
    )(edge_weight)


SR = 128


def _sym_body(lr_ref, lc_ref, m_ref, o_ref):
    s = pl.program_id(0)
    rr = SR * s + lax.broadcasted_iota(jnp.int32, (SR, N), 0)
    cc = lax.broadcasted_iota(jnp.int32, (SR, N), 1)
    upper = lc_ref[...].T
    val = jnp.where(cc <= rr, lr_ref[...], upper) * m_ref[...]
    o_ref[...] = val.reshape(SR * N)


def _symmetrize_mask(l_mat, mask):
    return pl.pallas_call(
        _sym_body,
        grid=(N // SR,),
        in_specs=[
            pl.BlockSpec((SR, N), lambda s: (s, 0)),
            pl.BlockSpec((N, SR), lambda s: (0, s)),
            pl.BlockSpec((SR, N), lambda s: (s, 0)),
        ],
        out_specs=pl.BlockSpec((SR * N,), lambda s: (s,)),
        out_shape=jax.ShapeDtypeStruct((N * N,), jnp.float32),
    )(l_mat, l_mat, mask)


@jax.jit
def kernel(mask, edge_weight, xs, ys):
    l_mat = _densify(edge_weight)
    return _symmetrize_mask(l_mat, mask)


# speedup vs baseline: 200.9965x; 1.5010x over previous
import functools

import jax
import jax.numpy as jnp
from jax import lax
from jax.experimental import pallas as pl
from jax.experimental.pallas import tpu as pltpu
from jax.experimental.pallas import tpu_sc as plsc

N = 4096
TOTAL = N * (N + 1) // 2
NC = 2
NS = 16
NW = NC * NS
ROWS_PER_W = N // NW
CH = 4128


def _densify_body(
    ew_hbm, l_hbm, chunk0, chunk1, row0, row1, sem_i0, sem_i1, sem_o0, sem_o1
):
    wid = lax.axis_index("s") * NC + lax.axis_index("c")
    iota16 = lax.iota(jnp.int32, 16)
    chunks = (chunk0, chunk1)
    rows = (row0, row1)
    sems_i = (sem_i0, sem_i1)
    sems_o = (sem_o0, sem_o1)

    def chunk_start(k):
        i = wid + NW * k
        t = (i * (i + 1)) // 2
        s = jnp.minimum(t - lax.rem(t, 8), TOTAL - CH)
        return pl.multiple_of(s, 8), t - s, i

    def start_fetch(k, p):
        s, _, _ = chunk_start(k)
        pltpu.make_async_copy(ew_hbm.at[pl.ds(s, CH)], chunks[p], sems_i[p]).start()

    start_fetch(0, 0)

    def do_row(k, p, prefetch_pred, drain_pred):
        s, r, i = chunk_start(k)

        @pl.when(prefetch_pred)
        def _():
            start_fetch(k + 1, 1 - p)

        pltpu.make_async_copy(ew_hbm.at[pl.ds(s, CH)], chunks[p], sems_i[p]).wait()

        @pl.when(drain_pred)
        def _():
            _, _, ip = chunk_start(k - 2)
            pltpu.make_async_copy(rows[p], l_hbm.at[ip], sems_o[p]).wait()

        n4 = (i + 64) // 64

        def shift_body(q4, _):
            base = 64 * q4 + r
            for u in range(4):
                v = plsc.load_gather(chunks[p], [base + 16 * u + iota16])
                rows[p][pl.ds(64 * q4 + 16 * u, 16)] = v
            return 0

        lax.fori_loop(0, n4, shift_body, 0)
        pltpu.make_async_copy(rows[p], l_hbm.at[i], sems_o[p]).start()

    def row_pair(k2, _):
        k = 2 * k2
        do_row(k, 0, k + 1 < ROWS_PER_W, k >= 2)
        do_row(k + 1, 1, k + 2 < ROWS_PER_W, k + 1 >= 2)
        return 0

    lax.fori_loop(0, ROWS_PER_W // 2, row_pair, 0)

    for k in (ROWS_PER_W - 2, ROWS_PER_W - 1):
        _, _, ip = chunk_start(k)
        pltpu.make_async_copy(rows[k % 2], l_hbm.at[ip], sems_o[k % 2]).wait()


def _densify(edge_weight):
    mesh = plsc.VectorSubcoreMesh(
        core_axis_name="c", subcore_axis_name="s", num_cores=NC, num_subcores=NS
    )
    return pl.kernel(
        _densify_body,
        out_type=jax.ShapeDtypeStruct((N, N), jnp.float32),
        mesh=mesh,
        compiler_params=pltpu.CompilerParams(needs_layout_passes=False),
        scratch_types=[
            pltpu.VMEM((CH,), jnp.float32),
            pltpu.VMEM((CH,), jnp.float32),
            pltpu.VMEM((N,), jnp.float32),
            pltpu.VMEM((N,), jnp.float32),
            pltpu.SemaphoreType.DMA,
            pltpu.SemaphoreType.DMA,
            pltpu.SemaphoreType.DMA,
            pltpu.SemaphoreType.DMA,
        ],
    )(edge_weight)


SR = 128


def _sym_body(lr_ref, lc_ref, m_ref, o_ref):
    s = pl.program_id(0)
    rr = SR * s + lax.broadcasted_iota(jnp.int32, (SR, N), 0)
    cc = lax.broadcasted_iota(jnp.int32, (SR, N), 1)
    upper = lc_ref[...].T
    val = jnp.where(cc <= rr, lr_ref[...], upper) * m_ref[...]
    o_ref[...] = val.reshape(SR * N)


def _symmetrize_mask(l_mat, mask):
    return pl.pallas_call(
        _sym_body,
        grid=(N // SR,),
        in_specs=[
            pl.BlockSpec((SR, N), lambda s: (s, 0)),
            pl.BlockSpec((N, SR), lambda s: (0, s)),
            pl.BlockSpec((SR, N), lambda s: (s, 0)),
        ],
        out_specs=pl.BlockSpec((SR * N,), lambda s: (s,)),
        out_shape=jax.ShapeDtypeStruct((N * N,), jnp.float32),
    )(l_mat, l_mat, mask)


@jax.jit
def kernel(mask, edge_weight, xs, ys):
    l_mat = _densify(edge_weight)
    return _symmetrize_mask(l_mat, mask)


# speedup vs baseline: 214.5943x; 1.0677x over previous
import functools

import jax
import jax.numpy as jnp
from jax import lax
from jax.experimental import pallas as pl
from jax.experimental.pallas import tpu as pltpu
from jax.experimental.pallas import tpu_sc as plsc

N = 4096
TOTAL = N * (N + 1) // 2
NC = 2
NS = 16
NW = NC * NS
ROWS_PER_W = N // NW
PIECE = 1024
NPIECE = 5
CH = PIECE * NPIECE


def _densify_body(
    ew_hbm, l_hbm, chunk0, chunk1, row0, row1, sem_i0, sem_i1, sem_o0, sem_o1
):
    wid = lax.axis_index("s") * NC + lax.axis_index("c")
    iota16 = lax.iota(jnp.int32, 16)
    chunks = (chunk0, chunk1)
    rows = (row0, row1)
    sems_i = (sem_i0, sem_i1)
    sems_o = (sem_o0, sem_o1)

    def chunk_start(k):
        i = wid + NW * k
        t = (i * (i + 1)) // 2
        s = jnp.minimum(t - lax.rem(t, 8), TOTAL - CH)
        return pl.multiple_of(s, 8), t - s, i

    def fetch_pieces(k, p, start):
        s, r, i = chunk_start(k)
        need = r + i + 1
        for q in range(NPIECE):
            @pl.when(PIECE * q < need)
            def _():
                d = pltpu.make_async_copy(
                    ew_hbm.at[pl.ds(s + PIECE * q, PIECE)],
                    chunks[p].at[pl.ds(PIECE * q, PIECE)],
                    sems_i[p],
                )
                d.start() if start else d.wait()

    def write_pieces(k, p, start):
        _, _, i = chunk_start(k)
        for q in range(NPIECE - 1):
            @pl.when(PIECE * q < i + 1)
            def _():
                d = pltpu.make_async_copy(
                    rows[p].at[pl.ds(PIECE * q, PIECE)],
                    l_hbm.at[i, pl.ds(PIECE * q, PIECE)],
                    sems_o[p],
                )
                d.start() if start else d.wait()

    fetch_pieces(0, 0, True)

    def do_row(k, p, prefetch_pred, drain_pred):
        s, r, i = chunk_start(k)

        @pl.when(prefetch_pred)
        def _():
            fetch_pieces(k + 1, 1 - p, True)

        fetch_pieces(k, p, False)

        @pl.when(drain_pred)
        def _():
            write_pieces(k - 2, p, False)

        n4 = (i + 64) // 64

        def shift_body(q4, _):
            base = 64 * q4 + r
            for u in range(4):
                v = plsc.load_gather(chunks[p], [base + 16 * u + iota16])
                rows[p][pl.ds(64 * q4 + 16 * u, 16)] = v
            return 0

        lax.fori_loop(0, n4, shift_body, 0)
        write_pieces(k, p, True)

    def row_pair(k2, _):
        k = 2 * k2
        do_row(k, 0, k + 1 < ROWS_PER_W, k >= 2)
        do_row(k + 1, 1, k + 2 < ROWS_PER_W, k + 1 >= 2)
        return 0

    lax.fori_loop(0, ROWS_PER_W // 2, row_pair, 0)

    for k in (ROWS_PER_W - 2, ROWS_PER_W - 1):
        write_pieces(k, k % 2, False)


def _densify(edge_weight):
    mesh = plsc.VectorSubcoreMesh(
        core_axis_name="c", subcore_axis_name="s", num_cores=NC, num_subcores=NS
    )
    return pl.kernel(
        _densify_body,
        out_type=jax.ShapeDtypeStruct((N, N), jnp.float32),
        mesh=mesh,
        compiler_params=pltpu.CompilerParams(needs_layout_passes=False),
        scratch_types=[
            pltpu.VMEM((CH,), jnp.float32),
            pltpu.VMEM((CH,), jnp.float32),
            pltpu.VMEM((N,), jnp.float32),
            pltpu.VMEM((N,), jnp.float32),
            pltpu.SemaphoreType.DMA,
            pltpu.SemaphoreType.DMA,
            pltpu.SemaphoreType.DMA,
            pltpu.SemaphoreType.DMA,
        ],
    )(edge_weight)


SR = 128


def _sym_body(lr_ref, lc_ref, m_ref, o_ref):
    s = pl.program_id(0)
    rr = SR * s + lax.broadcasted_iota(jnp.int32, (SR, N), 0)
    cc = lax.broadcasted_iota(jnp.int32, (SR, N), 1)
    upper = lc_ref[...].T
    val = jnp.where(cc <= rr, lr_ref[...], upper) * m_ref[...]
    o_ref[...] = val.reshape(SR * N)


def _symmetrize_mask(l_mat, mask):
    return pl.pallas_call(
        _sym_body,
        grid=(N // SR,),
        in_specs=[
            pl.BlockSpec((SR, N), lambda s: (s, 0)),
            pl.BlockSpec((N, SR), lambda s: (0, s)),
            pl.BlockSpec((SR, N), lambda s: (s, 0)),
        ],
        out_specs=pl.BlockSpec((SR * N,), lambda s: (s,)),
        out_shape=jax.ShapeDtypeStruct((N * N,), jnp.float32),
    )(l_mat, l_mat, mask)


@jax.jit
def kernel(mask, edge_weight, xs, ys):
    l_mat = _densify(edge_weight)
    return _symmetrize_mask(l_mat, mask)


# speedup vs baseline: 236.3218x; 1.1012x over previous
import functools

import jax
import jax.numpy as jnp
from jax import lax
from jax.experimental import pallas as pl
from jax.experimental.pallas import tpu as pltpu
from jax.experimental.pallas import tpu_sc as plsc

N = 4096
TOTAL = N * (N + 1) // 2
NC = 2
NS = 16
NW = NC * NS
PIECE = 1024
NPIECE = 5
CH = PIECE * NPIECE
SR = 128

BOUNDS = (0, 1024, 2048, 2944, 3584, 4096)
G = len(BOUNDS) - 1


def _pad_w(hi):
    return ((hi + PIECE - 1) // PIECE) * PIECE


def _densify_body(lo, hi, ew_hbm, l_hbm, chunk0, chunk1, row0, row1,
                  sem_i0, sem_i1, sem_o0, sem_o1):
    wid = lax.axis_index("s") * NC + lax.axis_index("c")
    iota16 = lax.iota(jnp.int32, 16)
    chunks = (chunk0, chunk1)
    rows = (row0, row1)
    sems_i = (sem_i0, sem_i1)
    sems_o = (sem_o0, sem_o1)
    cnt = (hi - lo) // NW

    def chunk_start(k):
        i = lo + wid + NW * k
        t = (i * (i + 1)) // 2
        s = jnp.minimum(t - lax.rem(t, 8), TOTAL - CH)
        return pl.multiple_of(s, 8), t - s, i

    def fetch_pieces(k, p, start):
        s, r, i = chunk_start(k)
        need = r + i + 1
        for q in range(NPIECE):
            @pl.when(PIECE * q < need)
            def _():
                d = pltpu.make_async_copy(
                    ew_hbm.at[pl.ds(s + PIECE * q, PIECE)],
                    chunks[p].at[pl.ds(PIECE * q, PIECE)],
                    sems_i[p],
                )
                d.start() if start else d.wait()

    def write_pieces(k, p, start):
        _, _, i = chunk_start(k)
        for q in range(_pad_w(hi) // PIECE):
            @pl.when(PIECE * q < i + 1)
            def _():
                d = pltpu.make_async_copy(
                    rows[p].at[pl.ds(PIECE * q, PIECE)],
                    l_hbm.at[i - lo, pl.ds(PIECE * q, PIECE)],
                    sems_o[p],
                )
                d.start() if start else d.wait()

    fetch_pieces(0, 0, True)

    def do_row(k, p, prefetch_pred, drain_pred):
        _, r, i = chunk_start(k)

        @pl.when(prefetch_pred)
        def _():
            fetch_pieces(k + 1, 1 - p, True)

        fetch_pieces(k, p, False)

        @pl.when(drain_pred)
        def _():
            write_pieces(k - 2, p, False)

        n4 = (i + 64) // 64

        def shift_body(q4, _):
            base = 64 * q4 + r
            for u in range(4):
                v = plsc.load_gather(chunks[p], [base + 16 * u + iota16])
                rows[p][pl.ds(64 * q4 + 16 * u, 16)] = v
            return 0

        lax.fori_loop(0, n4, shift_body, 0)
        write_pieces(k, p, True)

    def row_pair(k2, _):
        k = 2 * k2
        do_row(k, 0, k + 1 < cnt, k >= 2)
        do_row(k + 1, 1, k + 2 < cnt, k + 1 >= 2)
        return 0

    lax.fori_loop(0, cnt // 2, row_pair, 0)

    for k in (cnt - 2, cnt - 1):
        write_pieces(k, k % 2, False)


def _densify_group(g, edge_weight):
    lo, hi = BOUNDS[g], BOUNDS[g + 1]
    mesh = plsc.VectorSubcoreMesh(
        core_axis_name="c", subcore_axis_name="s", num_cores=NC, num_subcores=NS
    )
    return pl.kernel(
        functools.partial(_densify_body, lo, hi),
        out_type=jax.ShapeDtypeStruct((hi - lo, _pad_w(hi)), jnp.float32),
        mesh=mesh,
        compiler_params=pltpu.CompilerParams(needs_layout_passes=False),
        scratch_types=[
            pltpu.VMEM((CH,), jnp.float32),
            pltpu.VMEM((CH,), jnp.float32),
            pltpu.VMEM((N,), jnp.float32),
            pltpu.VMEM((N,), jnp.float32),
            pltpu.SemaphoreType.DMA,
            pltpu.SemaphoreType.DMA,
            pltpu.SemaphoreType.DMA,
            pltpu.SemaphoreType.DMA,
        ],
    )(edge_weight)


def _sym_body(g, lr_ref, *rest):
    lc_refs = rest[: G - g]
    m_ref = rest[G - g]
    o_ref = rest[-1]
    lo, hi = BOUNDS[g], BOUNDS[g + 1]
    s_glob = lo // SR + pl.program_id(0)
    lr = lr_ref[...]
    m = m_ref[...]
    pieces = []
    for h in range(G):
        lo_h, hi_h = BOUNDS[h], BOUNDS[h + 1]
        w = hi_h - lo_h
        mh = m[:, lo_h:hi_h]
        if h < g:
            pieces.append(lr[:, lo_h:hi_h] * mh)
        elif h == g:
            lct = lc_refs[0][...].T
            rr = SR * s_glob + lax.broadcasted_iota(jnp.int32, (SR, w), 0)
            cc = lo_h + lax.broadcasted_iota(jnp.int32, (SR, w), 1)
            pieces.append(jnp.where(cc <= rr, lr[:, lo_h:hi_h], lct) * mh)
        else:
            pieces.append(lc_refs[h - g][...].T * mh)
    val = jnp.concatenate(pieces, axis=1)
    o_ref[...] = val.reshape(SR * N)


def _sym_group(g, l_groups, mask, out_prev):
    lo, hi = BOUNDS[g], BOUNDS[g + 1]
    n_stripes = (hi - lo) // SR
    sb = lo // SR
    in_specs = [pl.BlockSpec((SR, hi), lambda s: (s, 0))]
    for h in range(g, G):
        rows_h = BOUNDS[h + 1] - BOUNDS[h]
        in_specs.append(
            pl.BlockSpec((rows_h, SR), lambda s, _sb=sb: (0, _sb + s))
        )
    in_specs.append(pl.BlockSpec((SR, N), lambda s, _sb=sb: (_sb + s, 0)))
    args = [l_groups[g], *l_groups[g:], mask]
    aliases = {}
    if out_prev is not None:
        in_specs.append(pl.BlockSpec(memory_space=pl.ANY))
        args.append(out_prev)
        aliases = {len(in_specs) - 1: 0}
    return pl.pallas_call(
        functools.partial(_sym_body, g),
        grid=(n_stripes,),
        in_specs=in_specs,
        out_specs=pl.BlockSpec((SR * N,), lambda s, _sb=sb: (_sb + s,)),
        out_shape=jax.ShapeDtypeStruct((N * N,), jnp.float32),
        input_output_aliases=aliases,
    )(*args)


@jax.jit
def kernel(mask, edge_weight, xs, ys):
    l_groups = [None] * G
    out = None
    for g in range(G - 1, -1, -1):
        l_groups[g] = _densify_group(g, edge_weight)
        out = _sym_group(g, l_groups, mask, out)
    return out
